# baseline (speedup 1.0000x reference)
import functools
import jax
import jax.numpy as jnp
from jax import lax
from jax.experimental import pallas as pl
from jax.experimental.pallas import tpu as pltpu
from jax.experimental.pallas import tpu_sc as plsc

D = 256
NX = NY = NZ = 32
B = 4
NC = 2
NS = 16
YC = 4
NCHUNK = NY // YC
NBUF = 2
NL = 16


def _body(xe_hbm, ye_hbm, ze_hbm, out_hbm, xev, yev, zev, xey, buf, sems):
    wid = lax.axis_index("s") * NC + lax.axis_index("c")
    x = wid

    pltpu.sync_copy(xe_hbm.at[x], xev)
    pltpu.sync_copy(ye_hbm.at[pl.ds(0, NY)], yev)
    pltpu.sync_copy(ze_hbm.at[pl.ds(0, NZ)], zev)

    for j in range(NCHUNK):
        slot = j % NBUF
        if j >= NBUF:
            for bb in range(B):
                pltpu.make_async_copy(
                    buf.at[slot],
                    out_hbm.at[bb, x, pl.ds((j - NBUF) * YC, YC)],
                    sems.at[slot, bb]).wait()

        for yy in range(YC):
            for c in range(D // NL):
                sl = pl.ds(c * NL, NL)
                xey[yy, sl] = xev[sl] + yev[j * YC + yy, sl]

        def z_step(z, _):
            for c in range(D // NL):
                sl = pl.ds(c * NL, NL)
                zv = zev[z, sl]
                for yy in range(YC):
                    buf[slot, yy, z, sl] = xey[yy, sl] + zv
            return 0

        lax.fori_loop(0, NZ, z_step, 0)

        for bb in range(B):
            pltpu.make_async_copy(
                buf.at[slot],
                out_hbm.at[bb, x, pl.ds(j * YC, YC)],
                sems.at[slot, bb]).start()

    for j in range(NCHUNK - NBUF, NCHUNK):
        slot = j % NBUF
        for bb in range(B):
            pltpu.make_async_copy(
                buf.at[slot],
                out_hbm.at[bb, x, pl.ds(j * YC, YC)],
                sems.at[slot, bb]).wait()


@functools.partial(jax.jit, static_argnames=())
def _sc_call(xe, ye, ze):
    mesh = plsc.VectorSubcoreMesh(core_axis_name="c", subcore_axis_name="s")
    return pl.kernel(
        _body,
        out_type=jax.ShapeDtypeStruct((B, NX, NY, NZ, D), jnp.float32),
        mesh=mesh,
        scratch_types=[
            pltpu.VMEM((D,), jnp.float32),
            pltpu.VMEM((NY, D), jnp.float32),
            pltpu.VMEM((NZ, D), jnp.float32),
            pltpu.VMEM((YC, D), jnp.float32),
            pltpu.VMEM((NBUF, YC, NZ, D), jnp.float32),
            pltpu.SemaphoreType.DMA((NBUF, B)),
        ],
    )(xe, ye, ze)


def kernel(features, x_embed, y_embed, z_embed):
    out = _sc_call(x_embed, y_embed, z_embed)
    return jnp.transpose(out, (0, 4, 1, 2, 3))


# speedup vs baseline: 1.5350x; 1.5350x over previous
import functools
import jax
import jax.numpy as jnp
from jax import lax
from jax.experimental import pallas as pl
from jax.experimental.pallas import tpu as pltpu
from jax.experimental.pallas import tpu_sc as plsc

D = 256
NX = NY = NZ = 32
B = 4
NC = 2
NS = 16
YC = 4
NCHUNK = NY // YC
NBUF = 2
NL = 16


def _body(xe_hbm, ye_hbm, ze_hbm, out_hbm, xev, yev, zev, xey, buf, sems):
    wid = lax.axis_index("s") * NC + lax.axis_index("c")
    x = wid

    pltpu.sync_copy(xe_hbm.at[x], xev)
    pltpu.sync_copy(ye_hbm.at[pl.ds(0, NY)], yev)
    pltpu.sync_copy(ze_hbm.at[pl.ds(0, NZ)], zev)

    for j in range(NCHUNK):
        slot = j % NBUF
        if j >= NBUF:
            for bb in range(B):
                pltpu.make_async_copy(
                    buf.at[slot],
                    out_hbm.at[bb, x, pl.ds((j - NBUF) * YC, YC)],
                    sems.at[slot, bb]).wait()

        for yy in range(YC):
            for c in range(D // NL):
                sl = pl.ds(c * NL, NL)
                xey[yy, sl] = xev[sl] + yev[j * YC + yy, sl]

        for c in range(D // NL):
            sl = pl.ds(c * NL, NL)
            xv = [xey[yy, sl] for yy in range(YC)]

            def z_step(z, _, sl=sl, xv=xv):
                zv = zev[z, sl]
                for yy in range(YC):
                    buf[slot, yy, z, sl] = xv[yy] + zv
                return 0

            lax.fori_loop(0, NZ, z_step, 0)

        for bb in range(B):
            pltpu.make_async_copy(
                buf.at[slot],
                out_hbm.at[bb, x, pl.ds(j * YC, YC)],
                sems.at[slot, bb]).start()

    for j in range(NCHUNK - NBUF, NCHUNK):
        slot = j % NBUF
        for bb in range(B):
            pltpu.make_async_copy(
                buf.at[slot],
                out_hbm.at[bb, x, pl.ds(j * YC, YC)],
                sems.at[slot, bb]).wait()


@functools.partial(jax.jit, static_argnames=())
def _sc_call(xe, ye, ze):
    mesh = plsc.VectorSubcoreMesh(core_axis_name="c", subcore_axis_name="s")
    return pl.kernel(
        _body,
        out_type=jax.ShapeDtypeStruct((B, NX, NY, NZ, D), jnp.float32),
        mesh=mesh,
        scratch_types=[
            pltpu.VMEM((D,), jnp.float32),
            pltpu.VMEM((NY, D), jnp.float32),
            pltpu.VMEM((NZ, D), jnp.float32),
            pltpu.VMEM((YC, D), jnp.float32),
            pltpu.VMEM((NBUF, YC, NZ, D), jnp.float32),
            pltpu.SemaphoreType.DMA((NBUF, B)),
        ],
    )(xe, ye, ze)


def kernel(features, x_embed, y_embed, z_embed):
    out = _sc_call(x_embed, y_embed, z_embed)
    return jnp.transpose(out, (0, 4, 1, 2, 3))
